# Initial kernel scaffold; baseline (speedup 1.0000x reference)
import jax
import jax.numpy as jnp
from jax.experimental import pallas as pl


def kernel(im0, flow):
    raise NotImplementedError("write your pallas kernel here")


# speedup vs baseline: 16.6974x; 16.6974x over previous
import functools

import jax
import jax.numpy as jnp
from jax import lax
from jax.experimental import pallas as pl
from jax.experimental.pallas import tpu as pltpu, tpu_sc as plsc

B, C, H, W = 4, 3, 512, 512
HW = H * W
NC, NS = 2, 16
BPC = B // NC
ROWS_PER_TEC = H // NS
NSEG = 4 * W // 128


def _tec_body(im0_hbm, fx_hbm, fy_hbm, out_hbm,
              fxbuf, fybuf, chbuf, idxst, v0st, v1st, v2st, zbuf, shared):
    c = lax.axis_index("c")
    s = lax.axis_index("s")

    lane = lax.iota(jnp.int32, 16)
    lanef = lane.astype(jnp.float32)
    zero16 = jnp.zeros((16,), jnp.float32)

    def zb(j, _):
        zbuf[pl.ds(j * 16, 16)] = zero16
        return 0
    lax.fori_loop(0, 256, zb, 0)

    myz = s * (BPC * C * HW // NS)
    for j in range(BPC * C * HW // NS // 4096):
        pltpu.sync_copy(zbuf, shared.at[pl.ds(myz + j * 4096, 4096)])
    plsc.subcore_barrier()

    for b_loc in range(BPC):
        b = c * BPC + b_loc

        def row_body(r, _):
            row = s * ROWS_PER_TEC + r
            pix0 = b * HW + row * W
            pltpu.sync_copy(fx_hbm.at[pl.ds(pix0, W)], fxbuf)
            pltpu.sync_copy(fy_hbm.at[pl.ds(pix0, W)], fybuf)
            for cc in range(C):
                pltpu.sync_copy(
                    im0_hbm.at[pl.ds((b * C + cc) * HW + row * W, W)],
                    chbuf.at[pl.ds(cc * W, W)])

            gyf = row.astype(jnp.float32)

            def vreg_body(v, _):
                off = v * 16
                fxv = fxbuf[pl.ds(off, 16)]
                fyv = fybuf[pl.ds(off, 16)]
                x = lanef + off.astype(jnp.float32) + fxv
                y = gyf + fyv
                xtf = x.astype(jnp.int32).astype(jnp.float32)
                ytf = y.astype(jnp.int32).astype(jnp.float32)
                x0f = xtf - jnp.where(xtf > x, 1.0, 0.0)
                y0f = ytf - jnp.where(ytf > y, 1.0, 0.0)
                ax = x - x0f
                ay = y - y0f
                bx = 1.0 - ax
                by = 1.0 - ay
                x0 = x0f.astype(jnp.int32)
                y0 = y0f.astype(jnp.int32)
                x1 = x0 + 1
                y1 = y0 + 1
                wx0 = jnp.where((x0 >= 0) & (x0 < W), bx, 0.0)
                wx1 = jnp.where((x1 >= 0) & (x1 < W), ax, 0.0)
                wy0 = jnp.where((y0 >= 0) & (y0 < H), by, 0.0)
                wy1 = jnp.where((y1 >= 0) & (y1 < H), ay, 0.0)
                x0c = jnp.clip(x0, 0, W - 1)
                x1c = jnp.clip(x1, 0, W - 1)
                r0 = jnp.clip(y0, 0, H - 1) * W
                r1 = jnp.clip(y1, 0, H - 1) * W
                ch0 = chbuf[pl.ds(off, 16)]
                ch1 = chbuf[pl.ds(W + off, 16)]
                ch2 = chbuf[pl.ds(2 * W + off, 16)]
                corners = (
                    (r0 + x0c, wx0 * wy0),
                    (r0 + x1c, wx1 * wy0),
                    (r1 + x0c, wx0 * wy1),
                    (r1 + x1c, wx1 * wy1),
                )
                seg = v >> 3
                col = (v & 7) * 16
                for k, (idxk, wk) in enumerate(corners):
                    idxst[k * 4 + seg, pl.ds(col, 16)] = idxk
                    v0st[k * 4 + seg, pl.ds(col, 16)] = ch0 * wk
                    v1st[k * 4 + seg, pl.ds(col, 16)] = ch1 * wk
                    v2st[k * 4 + seg, pl.ds(col, 16)] = ch2 * wk
                return 0

            lax.fori_loop(0, W // 16, vreg_body, 0)

            for cc, vst in enumerate((v0st, v1st, v2st)):
                img = (b_loc * C + cc) * HW
                for seg in range(NSEG):
                    pltpu.sync_copy(
                        vst.at[seg],
                        shared.at[pl.ds(img, HW)].at[idxst.at[seg]],
                        add=True)
            return 0

        lax.fori_loop(0, ROWS_PER_TEC, row_body, 0)

    plsc.subcore_barrier()
    n = BPC * C * HW // NS
    pltpu.sync_copy(shared.at[pl.ds(s * n, n)],
                    out_hbm.at[pl.ds((c * BPC * C * HW) + s * n, n)])


def kernel(im0, flow):
    im0r = im0.reshape(B * C * HW)
    fx = flow[..., 0].reshape(B * HW)
    fy = flow[..., 1].reshape(B * HW)

    mesh = plsc.VectorSubcoreMesh(core_axis_name="c", subcore_axis_name="s",
                                  num_cores=NC, num_subcores=NS)
    out = pl.kernel(
        _tec_body,
        out_type=jax.ShapeDtypeStruct((B * C * HW,), jnp.float32),
        mesh=mesh,
        scratch_types=[
            pltpu.VMEM((W,), jnp.float32),
            pltpu.VMEM((W,), jnp.float32),
            pltpu.VMEM((C * W,), jnp.float32),
            pltpu.VMEM((NSEG, 128), jnp.int32),
            pltpu.VMEM((NSEG, 128), jnp.float32),
            pltpu.VMEM((NSEG, 128), jnp.float32),
            pltpu.VMEM((NSEG, 128), jnp.float32),
            pltpu.VMEM((4096,), jnp.float32),
            pltpu.VMEM_SHARED((BPC * C * HW,), jnp.float32),
        ],
    )(im0r, fx, fy)
    return out.reshape(B, C, H, W)


# speedup vs baseline: 47.6551x; 2.8540x over previous
import jax
import jax.numpy as jnp
from jax import lax
from jax.experimental import pallas as pl
from jax.experimental.pallas import tpu as pltpu, tpu_sc as plsc

B, C, H, W = 4, 3, 512, 512
HW = H * W
NC, NS = 2, 16
BPC = B // NC
ROWS_PER_TEC = H // NS
CROWS = 2
CPIX = CROWS * W
NCHUNK = ROWS_PER_TEC // CROWS
NSEG = 4 * CPIX // 128
VREGS = CPIX // 16
ZB = 2048
SLC = C * HW // NS


def _splat_chunk(ci, s, fxb, fyb, chb, idxst, v0, v1, v2, lanef):
    def vreg_body(v, _):
        off = v * 16
        rowin = v >> 5
        gxb = (v & 31) * 16
        fxv = fxb[pl.ds(off, 16)]
        fyv = fyb[pl.ds(off, 16)]
        x = lanef + gxb.astype(jnp.float32) + fxv
        row = s * ROWS_PER_TEC + ci * CROWS + rowin
        y = row.astype(jnp.float32) + fyv
        xtf = x.astype(jnp.int32).astype(jnp.float32)
        ytf = y.astype(jnp.int32).astype(jnp.float32)
        x0f = xtf - jnp.where(xtf > x, 1.0, 0.0)
        y0f = ytf - jnp.where(ytf > y, 1.0, 0.0)
        ax = x - x0f
        ay = y - y0f
        bx = 1.0 - ax
        by = 1.0 - ay
        x0 = x0f.astype(jnp.int32)
        y0 = y0f.astype(jnp.int32)
        x1 = x0 + 1
        y1 = y0 + 1
        wx0 = jnp.where((x0 >= 0) & (x0 < W), bx, 0.0)
        wx1 = jnp.where((x1 >= 0) & (x1 < W), ax, 0.0)
        wy0 = jnp.where((y0 >= 0) & (y0 < H), by, 0.0)
        wy1 = jnp.where((y1 >= 0) & (y1 < H), ay, 0.0)
        x0c = jnp.clip(x0, 0, W - 1)
        x1c = jnp.clip(x1, 0, W - 1)
        r0 = jnp.clip(y0, 0, H - 1) * W
        r1 = jnp.clip(y1, 0, H - 1) * W
        ch0 = chb[pl.ds(off, 16)]
        ch1 = chb[pl.ds(CPIX + off, 16)]
        ch2 = chb[pl.ds(2 * CPIX + off, 16)]
        corners = (
            (r0 + x0c, wx0 * wy0),
            (r0 + x1c, wx1 * wy0),
            (r1 + x0c, wx0 * wy1),
            (r1 + x1c, wx1 * wy1),
        )
        seg = v >> 3
        col = (v & 7) * 16
        for k, (idxk, wk) in enumerate(corners):
            pos = k * CPIX + off
            idxst[k * (NSEG // 4) + seg, pl.ds(col, 16)] = idxk
            v0[pl.ds(pos, 16)] = ch0 * wk
            v1[pl.ds(pos, 16)] = ch1 * wk
            v2[pl.ds(pos, 16)] = ch2 * wk
        return 0

    lax.fori_loop(0, VREGS, vreg_body, 0)


def _tec_body(im0_hbm, fx_hbm, fy_hbm, out_hbm,
              fxb, fyb, chb, idxst, v0, v1, v2, zbuf, shared,
              sin, ssc):
    c = lax.axis_index("c")
    s = lax.axis_index("s")

    lane = lax.iota(jnp.int32, 16)
    lanef = lane.astype(jnp.float32)
    zero16 = jnp.zeros((16,), jnp.float32)

    def fire_inputs(b, ci, p):
        pix0 = b * HW + (s * ROWS_PER_TEC + ci * CROWS) * W
        pltpu.async_copy(fx_hbm.at[pl.ds(pix0, CPIX)], fxb[p], sin[p])
        pltpu.async_copy(fy_hbm.at[pl.ds(pix0, CPIX)], fyb[p], sin[p])
        for cc in range(C):
            pltpu.async_copy(
                im0_hbm.at[pl.ds((b * C + cc) * HW
                                 + (s * ROWS_PER_TEC + ci * CROWS) * W,
                                 CPIX)],
                chb[p].at[pl.ds(cc * CPIX, CPIX)], sin[p])

    def drain_inputs(p):
        pltpu.make_async_copy(fx_hbm.at[pl.ds(0, CPIX)], fxb[p],
                              sin[p]).wait()
        pltpu.make_async_copy(fy_hbm.at[pl.ds(0, CPIX)], fyb[p],
                              sin[p]).wait()
        pltpu.make_async_copy(im0_hbm.at[pl.ds(0, C * CPIX)], chb[p],
                              sin[p]).wait()

    def fire_scatters(p):
        for cc, vst in enumerate((v0[p], v1[p], v2[p])):
            img = cc * HW
            for seg in range(NSEG):
                pltpu.async_copy(
                    vst.at[pl.ds(seg * 128, 128)],
                    shared.at[pl.ds(img, HW)].at[idxst[p].at[seg]],
                    ssc[p], add=True)

    def drain_scatters(p):
        for vst in (v0[p], v1[p], v2[p]):
            pltpu.make_async_copy(fx_hbm.at[pl.ds(0, 4 * CPIX)], vst,
                                  ssc[p]).wait()

    def zero_slice():
        for j in range(SLC // ZB):
            pltpu.sync_copy(zbuf, shared.at[pl.ds(s * SLC + j * ZB, ZB)])

    def zb(j, _):
        zbuf[pl.ds(j * 16, 16)] = zero16
        return 0
    lax.fori_loop(0, ZB // 16, zb, 0)

    for b_loc in range(BPC):
        b = c * BPC + b_loc
        fire_inputs(b, jnp.int32(0), 0)
        if b_loc == 0:
            zero_slice()
        plsc.subcore_barrier()

        def pair_body(pair, _):
            for p in range(2):
                ci = pair * 2 + p
                drain_inputs(p)

                @pl.when(ci < NCHUNK - 1)
                def _():
                    fire_inputs(b, ci + 1, 1 - p)

                @pl.when(pair >= 1)
                def _():
                    drain_scatters(p)

                _splat_chunk(ci, s, fxb[p], fyb[p], chb[p],
                             idxst[p], v0[p], v1[p], v2[p], lanef)
                fire_scatters(p)
            return 0

        lax.fori_loop(0, NCHUNK // 2, pair_body, 0)
        for p in range(2):
            drain_scatters(p)

        plsc.subcore_barrier()
        pltpu.sync_copy(
            shared.at[pl.ds(s * SLC, SLC)],
            out_hbm.at[pl.ds(b * C * HW + s * SLC, SLC)])
        if b_loc < BPC - 1:
            zero_slice()


def kernel(im0, flow):
    im0r = im0.reshape(B * C * HW)
    fx = flow[..., 0].reshape(B * HW)
    fy = flow[..., 1].reshape(B * HW)

    mesh = plsc.VectorSubcoreMesh(core_axis_name="c", subcore_axis_name="s",
                                  num_cores=NC, num_subcores=NS)
    dbl = lambda t: [t, t]
    out = pl.kernel(
        _tec_body,
        out_type=jax.ShapeDtypeStruct((B * C * HW,), jnp.float32),
        mesh=mesh,
        scratch_types=[
            dbl(pltpu.VMEM((CPIX,), jnp.float32)),
            dbl(pltpu.VMEM((CPIX,), jnp.float32)),
            dbl(pltpu.VMEM((C * CPIX,), jnp.float32)),
            dbl(pltpu.VMEM((NSEG, 128), jnp.int32)),
            dbl(pltpu.VMEM((4 * CPIX,), jnp.float32)),
            dbl(pltpu.VMEM((4 * CPIX,), jnp.float32)),
            dbl(pltpu.VMEM((4 * CPIX,), jnp.float32)),
            pltpu.VMEM((ZB,), jnp.float32),
            pltpu.VMEM_SHARED((C * HW,), jnp.float32),
            dbl(pltpu.SemaphoreType.DMA),
            dbl(pltpu.SemaphoreType.DMA),
        ],
    )(im0r, fx, fy)
    return out.reshape(B, C, H, W)
